# baseline (speedup 1.0000x reference)
import functools

import jax
import jax.numpy as jnp
from jax import lax
from jax.experimental import pallas as pl
from jax.experimental.pallas import tpu as pltpu
from jax.experimental.pallas import tpu_sc as plsc

B = 16384
NN = 13
NC = 26
D = 32
V = 100000

NUM_CORES = 2
NUM_SUBCORES = 16
NW = NUM_CORES * NUM_SUBCORES
BPW = B // NW
CB = 64
NCHUNK = BPW // CB
IDX_CHUNK = CB * NC
PAT = 208


def _body(emb_hbm, idx_hbm, xn_hbm, w_hbm, bias_hbm, pat_hbm,
          num_out, cat_out,
          idx_v, rows_v, x_v, numst_v, w_v, bias_v, pat_v, sem):
    wid = lax.axis_index("s") * NUM_CORES + lax.axis_index("c")
    base_b = wid * BPW

    pltpu.sync_copy(w_hbm, w_v)
    pltpu.sync_copy(bias_hbm, bias_v)
    pltpu.sync_copy(pat_hbm, pat_v)

    def chunk(c, _):
        b0 = base_b + c * CB
        pltpu.sync_copy(idx_hbm.at[pl.ds(b0 * NC, IDX_CHUNK)], idx_v)

        def add_offs(g, _):
            for vv in range(PAT // 16):
                sl = pl.ds(g * PAT + vv * 16, 16)
                idx_v[sl] = idx_v[sl] + pat_v[pl.ds(vv * 16, 16)]
            return 0

        lax.fori_loop(0, IDX_CHUNK // PAT, add_offs, 0)

        gat = pltpu.async_copy(emb_hbm.at[idx_v], rows_v, sem)

        pltpu.sync_copy(xn_hbm.at[pl.ds(b0, CB), :], x_v)

        def nrow(b, _):
            xrow = x_v[b]
            for n in range(NN):
                xs = xrow[n]
                for h in range(D // 16):
                    sl = pl.ds(h * 16, 16)
                    numst_v[b, n, sl] = xs * w_v[n, sl] + bias_v[n, sl]
            return 0

        lax.fori_loop(0, CB, nrow, 0)
        pltpu.sync_copy(numst_v, num_out.at[pl.ds(b0, CB)])

        gat.wait()
        pltpu.sync_copy(rows_v, cat_out.at[pl.ds(b0 * NC, IDX_CHUNK)])
        return 0

    lax.fori_loop(0, NCHUNK, chunk, 0)


@jax.jit
def _run(emb2, idxflat, xn2, w2, bias2, pat):
    mesh = plsc.VectorSubcoreMesh(
        core_axis_name="c", subcore_axis_name="s",
        num_cores=NUM_CORES, num_subcores=NUM_SUBCORES)
    f = pl.kernel(
        _body,
        out_type=(
            jax.ShapeDtypeStruct((B, NN, D), jnp.float32),
            jax.ShapeDtypeStruct((B * NC, D), jnp.float32),
        ),
        mesh=mesh,
        scratch_types=[
            pltpu.VMEM((IDX_CHUNK,), jnp.int32),
            pltpu.VMEM((IDX_CHUNK, D), jnp.float32),
            pltpu.VMEM((CB, 16), jnp.float32),
            pltpu.VMEM((CB, NN, D), jnp.float32),
            pltpu.VMEM((NN, D), jnp.float32),
            pltpu.VMEM((NN, D), jnp.float32),
            pltpu.VMEM((PAT,), jnp.int32),
            pltpu.SemaphoreType.DMA,
        ],
        compiler_params=pltpu.CompilerParams(use_tc_tiling_on_sc=False),
    )
    return f(emb2, idxflat, xn2, w2, bias2, pat)


def kernel(x_numeric, x_categorical, W_num, b_num, emb):
    emb2 = emb.reshape(NC * V, D)
    idxflat = x_categorical.reshape(B * NC)
    xn2 = jnp.pad(x_numeric.reshape(B, NN), ((0, 0), (0, 16 - NN)))
    w2 = W_num.reshape(NN, D)
    pat = (jnp.arange(PAT, dtype=jnp.int32) % NC) * V
    num, cat = _run(emb2, idxflat, xn2, w2, b_num, pat)
    merged = jnp.concatenate(
        [num.reshape(B, 1, NN * D), cat.reshape(B, 1, NC * D)], axis=2)
    return merged


# speedup vs baseline: 3.0126x; 3.0126x over previous
import jax
import jax.numpy as jnp
from jax import lax
from jax.experimental import pallas as pl
from jax.experimental.pallas import tpu as pltpu
from jax.experimental.pallas import tpu_sc as plsc

B = 16384
NN = 13
NC = 26
D = 32
V = 100000

NUM_CORES = 2
NUM_SUBCORES = 16
CHUNK = 4096
NCH = B // CHUNK


def _body(emb_hbm, idx_hbm, xn_hbm, w_hbm, bias_hbm, out_hbm,
          plane_v, ich_v, rch_v, xch_v, w_v, bias_v):
    d = lax.axis_index("s") * NUM_CORES + lax.axis_index("c")

    pltpu.sync_copy(w_hbm, w_v)
    pltpu.sync_copy(bias_hbm, bias_v)
    d16 = jnp.full((16,), d, dtype=jnp.int32)

    def cat_task(f, _):
        pltpu.sync_copy(emb_hbm.at[f, d], plane_v)
        r = NN * D + f * D + d

        def chunk(c, _):
            b0 = c * CHUNK
            pltpu.sync_copy(idx_hbm.at[pl.ds(f * B + b0, CHUNK)], ich_v)

            def grp(g, _):
                sl = pl.ds(g * 16, 16)
                rch_v[sl] = plsc.load_gather(plane_v, [ich_v[sl]])
                return 0

            lax.fori_loop(0, CHUNK // 16, grp, 0)
            pltpu.sync_copy(rch_v, out_hbm.at[r, pl.ds(b0, CHUNK)])
            return 0

        lax.fori_loop(0, NCH, chunk, 0)
        return 0

    lax.fori_loop(0, NC, cat_task, 0)

    def num_task(n, _):
        wb = plsc.load_gather(w_v, [n * D + d16])
        bb = plsc.load_gather(bias_v, [n * D + d16])
        r = n * D + d

        def chunk(c, _):
            b0 = c * CHUNK
            pltpu.sync_copy(xn_hbm.at[pl.ds(n * B + b0, CHUNK)], xch_v)

            def grp(g, _):
                sl = pl.ds(g * 16, 16)
                rch_v[sl] = xch_v[sl] * wb + bb
                return 0

            lax.fori_loop(0, CHUNK // 16, grp, 0)
            pltpu.sync_copy(rch_v, out_hbm.at[r, pl.ds(b0, CHUNK)])
            return 0

        lax.fori_loop(0, NCH, chunk, 0)
        return 0

    lax.fori_loop(0, NN, num_task, 0)


@jax.jit
def _run(emb_t, idx_t, xn_t, w1, bias1):
    mesh = plsc.VectorSubcoreMesh(
        core_axis_name="c", subcore_axis_name="s",
        num_cores=NUM_CORES, num_subcores=NUM_SUBCORES)
    f = pl.kernel(
        _body,
        out_type=jax.ShapeDtypeStruct(((NN + NC) * D, B), jnp.float32),
        mesh=mesh,
        scratch_types=[
            pltpu.VMEM((V,), jnp.float32),
            pltpu.VMEM((CHUNK,), jnp.int32),
            pltpu.VMEM((CHUNK,), jnp.float32),
            pltpu.VMEM((CHUNK,), jnp.float32),
            pltpu.VMEM((NN * D,), jnp.float32),
            pltpu.VMEM((NN * D,), jnp.float32),
        ],
        compiler_params=pltpu.CompilerParams(needs_layout_passes=False),
    )
    return f(emb_t, idx_t, xn_t, w1, bias1)


def kernel(x_numeric, x_categorical, W_num, b_num, emb):
    emb_t = jnp.transpose(emb, (0, 2, 1))
    idx_t = jnp.transpose(x_categorical[:, :, 0], (1, 0)).reshape(NC * B)
    xn_t = jnp.transpose(x_numeric[:, :, 0], (1, 0)).reshape(NN * B)
    w1 = W_num.reshape(NN * D)
    bias1 = b_num.reshape(NN * D)
    out = _run(emb_t, idx_t, xn_t, w1, bias1)
    return jnp.transpose(out, (1, 0)).reshape(B, 1, (NN + NC) * D)


# speedup vs baseline: 3.1048x; 1.0306x over previous
import jax
import jax.numpy as jnp
from jax import lax
from jax.experimental import pallas as pl
from jax.experimental.pallas import tpu as pltpu
from jax.experimental.pallas import tpu_sc as plsc

B = 16384
NN = 13
NC = 26
D = 32
V = 100000

NUM_CORES = 2
NUM_SUBCORES = 16
CHUNK = 4096
NCH = B // CHUNK


def _body(emb_hbm, idx_hbm, xn_hbm, w_hbm, bias_hbm, out_hbm,
          plane_v, ich0_v, ich1_v, rch0_v, rch1_v, xch_v, w_v, bias_v,
          sem_i0, sem_i1, sem_o0, sem_o1):
    d = lax.axis_index("s") * NUM_CORES + lax.axis_index("c")
    ich = (ich0_v, ich1_v)
    rch = (rch0_v, rch1_v)
    sem_i = (sem_i0, sem_i1)
    sem_o = (sem_o0, sem_o1)

    pltpu.sync_copy(w_hbm, w_v)
    pltpu.sync_copy(bias_hbm, bias_v)
    d16 = jnp.full((16,), d, dtype=jnp.int32)

    def ich_src(f, c):
        return idx_hbm.at[pl.ds(f * B + c * CHUNK, CHUNK)]

    pltpu.async_copy(ich_src(0, 0), ich[0], sem_i[0])

    def cat_task(f, _):
        pltpu.sync_copy(emb_hbm.at[f, d], plane_v)
        r = NN * D + f * D + d

        for c in range(NCH):
            p = c % 2
            b0 = c * CHUNK
            pltpu.make_async_copy(ich_src(f, c), ich[p], sem_i[p]).wait()
            if c + 1 < NCH:
                nf, nc = f, c + 1
            else:
                nf, nc = jnp.minimum(f + 1, NC - 1), 0
            pltpu.async_copy(ich_src(nf, nc), ich[1 - p], sem_i[1 - p])

            dst = out_hbm.at[r, pl.ds(b0, CHUNK)]
            if c >= 2:
                pltpu.make_async_copy(rch[p], dst, sem_o[p]).wait()
            else:
                @pl.when(f > 0)
                def _():
                    pltpu.make_async_copy(rch[p], dst, sem_o[p]).wait()

            def grp(g, _):
                sl = pl.ds(g * 16, 16)
                rch[p][sl] = plsc.load_gather(plane_v, [ich[p][sl]])
                return 0

            lax.fori_loop(0, CHUNK // 16, grp, 0, unroll=8)
            pltpu.async_copy(rch[p], dst, sem_o[p])
        return 0

    lax.fori_loop(0, NC, cat_task, 0)

    last = NN * D + (NC - 1) * D + d
    pltpu.make_async_copy(ich_src(NC - 1, 0), ich[0], sem_i[0]).wait()
    pltpu.make_async_copy(
        rch[0], out_hbm.at[last, pl.ds((NCH - 2) * CHUNK, CHUNK)], sem_o[0]
    ).wait()
    pltpu.make_async_copy(
        rch[1], out_hbm.at[last, pl.ds((NCH - 1) * CHUNK, CHUNK)], sem_o[1]
    ).wait()

    def num_task(n, _):
        wb = plsc.load_gather(w_v, [n * D + d16])
        bb = plsc.load_gather(bias_v, [n * D + d16])
        r = n * D + d

        def chunk(c, _):
            b0 = c * CHUNK
            pltpu.sync_copy(xn_hbm.at[pl.ds(n * B + b0, CHUNK)], xch_v)

            def grp(g, _):
                sl = pl.ds(g * 16, 16)
                rch0_v[sl] = xch_v[sl] * wb + bb
                return 0

            lax.fori_loop(0, CHUNK // 16, grp, 0, unroll=8)
            pltpu.sync_copy(rch0_v, out_hbm.at[r, pl.ds(b0, CHUNK)])
            return 0

        lax.fori_loop(0, NCH, chunk, 0)
        return 0

    lax.fori_loop(0, NN, num_task, 0)


@jax.jit
def _run(emb_t, idx_t, xn_t, w1, bias1):
    mesh = plsc.VectorSubcoreMesh(
        core_axis_name="c", subcore_axis_name="s",
        num_cores=NUM_CORES, num_subcores=NUM_SUBCORES)
    f = pl.kernel(
        _body,
        out_type=jax.ShapeDtypeStruct(((NN + NC) * D, B), jnp.float32),
        mesh=mesh,
        scratch_types=[
            pltpu.VMEM((V,), jnp.float32),
            pltpu.VMEM((CHUNK,), jnp.int32),
            pltpu.VMEM((CHUNK,), jnp.int32),
            pltpu.VMEM((CHUNK,), jnp.float32),
            pltpu.VMEM((CHUNK,), jnp.float32),
            pltpu.VMEM((CHUNK,), jnp.float32),
            pltpu.VMEM((NN * D,), jnp.float32),
            pltpu.VMEM((NN * D,), jnp.float32),
            pltpu.SemaphoreType.DMA,
            pltpu.SemaphoreType.DMA,
            pltpu.SemaphoreType.DMA,
            pltpu.SemaphoreType.DMA,
        ],
        compiler_params=pltpu.CompilerParams(needs_layout_passes=False),
    )
    return f(emb_t, idx_t, xn_t, w1, bias1)


def kernel(x_numeric, x_categorical, W_num, b_num, emb):
    emb_t = jnp.transpose(emb, (0, 2, 1))
    idx_t = jnp.transpose(x_categorical[:, :, 0], (1, 0)).reshape(NC * B)
    xn_t = jnp.transpose(x_numeric[:, :, 0], (1, 0)).reshape(NN * B)
    w1 = W_num.reshape(NN * D)
    bias1 = b_num.reshape(NN * D)
    out = _run(emb_t, idx_t, xn_t, w1, bias1)
    return jnp.transpose(out, (1, 0)).reshape(B, 1, (NN + NC) * D)


# speedup vs baseline: 3.5831x; 1.1540x over previous
import jax
import jax.numpy as jnp
from jax import lax
from jax.experimental import pallas as pl
from jax.experimental.pallas import tpu as pltpu
from jax.experimental.pallas import tpu_sc as plsc

B = 16384
NN = 13
NC = 26
D = 32
V = 100000

NUM_CORES = 2
NUM_SUBCORES = 16
ICH = 8192
RCH = 4096


def _body(emb_hbm, idx_hbm, xn_hbm, w_hbm, bias_hbm, out_hbm,
          plane_v, ich0_v, ich1_v, rch0_v, rch1_v, w_v, bias_v,
          sem_i0, sem_i1, sem_o0, sem_o1, sem_x0, sem_x1):
    d = lax.axis_index("s") * NUM_CORES + lax.axis_index("c")
    ich = (ich0_v, ich1_v)
    rch = (rch0_v, rch1_v)
    sem_i = (sem_i0, sem_i1)
    sem_o = (sem_o0, sem_o1)
    sem_x = (sem_x0, sem_x1)

    pltpu.sync_copy(w_hbm, w_v)
    pltpu.sync_copy(bias_hbm, bias_v)
    d16 = jnp.full((16,), d, dtype=jnp.int32)

    def ich_src(f, h):
        return idx_hbm.at[pl.ds(f * B + h * ICH, ICH)]

    pltpu.async_copy(ich_src(0, 0), ich[0], sem_i[0])

    def cat_task(f, _):
        pltpu.sync_copy(emb_hbm.at[f, d], plane_v)
        r = NN * D + f * D + d

        for h in range(B // ICH):
            pltpu.make_async_copy(ich_src(f, h), ich[h % 2], sem_i[h % 2]).wait()
            if h + 1 < B // ICH:
                nf, nh = f, h + 1
            else:
                nf, nh = jnp.minimum(f + 1, NC - 1), 0
            pltpu.async_copy(ich_src(nf, nh), ich[1 - h % 2], sem_i[1 - h % 2])

            for q in range(ICH // RCH):
                c = h * (ICH // RCH) + q
                p = c % 2
                b0 = c * RCH
                dst = out_hbm.at[r, pl.ds(b0, RCH)]
                if c >= 2:
                    pltpu.make_async_copy(rch[p], dst, sem_o[p]).wait()
                else:
                    @pl.when(f > 0)
                    def _():
                        pltpu.make_async_copy(rch[p], dst, sem_o[p]).wait()

                def grp(g, _):
                    rch[p][pl.ds(g * 16, 16)] = plsc.load_gather(
                        plane_v, [ich[h % 2][pl.ds(q * RCH + g * 16, 16)]])
                    return 0

                lax.fori_loop(0, RCH // 16, grp, 0, unroll=8)
                pltpu.async_copy(rch[p], dst, sem_o[p])
        return 0

    lax.fori_loop(0, NC, cat_task, 0)

    last = NN * D + (NC - 1) * D + d
    nq = B // RCH
    pltpu.make_async_copy(ich_src(NC - 1, 0), ich[0], sem_i[0]).wait()
    pltpu.make_async_copy(
        rch[0], out_hbm.at[last, pl.ds((nq - 2) * RCH, RCH)], sem_o[0]
    ).wait()
    pltpu.make_async_copy(
        rch[1], out_hbm.at[last, pl.ds((nq - 1) * RCH, RCH)], sem_o[1]
    ).wait()

    def xn_src(n, c):
        return xn_hbm.at[pl.ds(n * B + c * RCH, RCH)]

    pltpu.async_copy(xn_src(0, 0), rch[0], sem_x[0])

    def num_task(n, _):
        wb = plsc.load_gather(w_v, [n * D + d16])
        bb = plsc.load_gather(bias_v, [n * D + d16])
        r = n * D + d

        for c in range(B // RCH):
            p = c % 2
            b0 = c * RCH
            dst = out_hbm.at[r, pl.ds(b0, RCH)]
            pltpu.make_async_copy(xn_src(n, c), rch[p], sem_x[p]).wait()
            if c + 1 < B // RCH:
                nn_, ncc = n, c + 1
            else:
                nn_, ncc = jnp.minimum(n + 1, NN - 1), 0

            def grp(g, _):
                sl = pl.ds(g * 16, 16)
                rch[p][sl] = rch[p][sl] * wb + bb
                return 0

            lax.fori_loop(0, RCH // 16, grp, 0, unroll=8)
            pltpu.async_copy(rch[p], dst, sem_o[p])
            if c >= 1:
                pltpu.make_async_copy(rch[1 - p], dst, sem_o[1 - p]).wait()
            else:
                @pl.when(n > 0)
                def _():
                    pltpu.make_async_copy(rch[1 - p], dst, sem_o[1 - p]).wait()
            pltpu.async_copy(xn_src(nn_, ncc), rch[1 - p], sem_x[1 - p])
        return 0

    lax.fori_loop(0, NN, num_task, 0)

    lastn = (NN - 1) * D + d
    pltpu.make_async_copy(xn_src(NN - 1, 0), rch[0], sem_x[0]).wait()
    pltpu.make_async_copy(
        rch[1], out_hbm.at[lastn, pl.ds((nq - 1) * RCH, RCH)], sem_o[1]
    ).wait()


@jax.jit
def _run(emb_t, idx_t, xn_t, w1, bias1):
    mesh = plsc.VectorSubcoreMesh(
        core_axis_name="c", subcore_axis_name="s",
        num_cores=NUM_CORES, num_subcores=NUM_SUBCORES)
    f = pl.kernel(
        _body,
        out_type=jax.ShapeDtypeStruct(((NN + NC) * D, B), jnp.float32),
        mesh=mesh,
        scratch_types=[
            pltpu.VMEM((V,), jnp.float32),
            pltpu.VMEM((ICH,), jnp.int32),
            pltpu.VMEM((ICH,), jnp.int32),
            pltpu.VMEM((RCH,), jnp.float32),
            pltpu.VMEM((RCH,), jnp.float32),
            pltpu.VMEM((NN * D,), jnp.float32),
            pltpu.VMEM((NN * D,), jnp.float32),
            pltpu.SemaphoreType.DMA,
            pltpu.SemaphoreType.DMA,
            pltpu.SemaphoreType.DMA,
            pltpu.SemaphoreType.DMA,
            pltpu.SemaphoreType.DMA,
            pltpu.SemaphoreType.DMA,
        ],
        compiler_params=pltpu.CompilerParams(needs_layout_passes=False),
    )
    return f(emb_t, idx_t, xn_t, w1, bias1)


def kernel(x_numeric, x_categorical, W_num, b_num, emb):
    emb_t = jnp.transpose(emb, (0, 2, 1))
    idx_t = jnp.transpose(x_categorical[:, :, 0], (1, 0)).reshape(NC * B)
    xn_t = jnp.transpose(x_numeric[:, :, 0], (1, 0)).reshape(NN * B)
    w1 = W_num.reshape(NN * D)
    bias1 = b_num.reshape(NN * D)
    out = _run(emb_t, idx_t, xn_t, w1, bias1)
    return jnp.transpose(out, (1, 0)).reshape(B, 1, (NN + NC) * D)


# speedup vs baseline: 5.8333x; 1.6280x over previous
import jax
import jax.numpy as jnp
from jax import lax
from jax.experimental import pallas as pl
from jax.experimental.pallas import tpu as pltpu
from jax.experimental.pallas import tpu_sc as plsc

B = 16384
NN = 13
NC = 26
D = 32
V = 100000

NUM_CORES = 2
NUM_SUBCORES = 16
ICH = 8192
RCH = 4096


def _body(emb_hbm, idx_hbm, xn_hbm, w_hbm, bias_hbm, out_hbm,
          plane_v, ich0_v, ich1_v, rch0_v, rch1_v, w_v, bias_v,
          sem_i0, sem_i1, sem_o0, sem_o1, sem_x0, sem_x1):
    d = lax.axis_index("s") * NUM_CORES + lax.axis_index("c")
    ich = (ich0_v, ich1_v)
    rch = (rch0_v, rch1_v)
    sem_i = (sem_i0, sem_i1)
    sem_o = (sem_o0, sem_o1)
    sem_x = (sem_x0, sem_x1)

    pltpu.sync_copy(w_hbm, w_v)
    pltpu.sync_copy(bias_hbm, bias_v)
    d16 = jnp.full((16,), d, dtype=jnp.int32)

    def ich_src(f, h):
        return idx_hbm.at[pl.ds(f * B + h * ICH, ICH)]

    pltpu.async_copy(ich_src(0, 0), ich[0], sem_i[0])

    def cat_task(f, _):
        pltpu.sync_copy(emb_hbm.at[f, d], plane_v)
        r = NN * D + f * D + d

        for h in range(B // ICH):
            pltpu.make_async_copy(ich_src(f, h), ich[h % 2], sem_i[h % 2]).wait()
            if h + 1 < B // ICH:
                nf, nh = f, h + 1
            else:
                nf, nh = jnp.minimum(f + 1, NC - 1), 0
            pltpu.async_copy(ich_src(nf, nh), ich[1 - h % 2], sem_i[1 - h % 2])

            for q in range(ICH // RCH):
                c = h * (ICH // RCH) + q
                p = c % 2
                b0 = c * RCH
                dst = out_hbm.at[r, pl.ds(b0, RCH)]
                if c >= 2:
                    pltpu.make_async_copy(rch[p], dst, sem_o[p]).wait()
                else:
                    @pl.when(f > 0)
                    def _():
                        pltpu.make_async_copy(rch[p], dst, sem_o[p]).wait()

                @plsc.parallel_loop(0, RCH, 16, unroll=8)
                def _(g):
                    rch[p][pl.ds(g, 16)] = plsc.load_gather(
                        plane_v, [ich[h % 2][pl.ds(q * RCH + g, 16)]])
                pltpu.async_copy(rch[p], dst, sem_o[p])
        return 0

    lax.fori_loop(0, NC, cat_task, 0)

    last = NN * D + (NC - 1) * D + d
    nq = B // RCH
    pltpu.make_async_copy(ich_src(NC - 1, 0), ich[0], sem_i[0]).wait()
    pltpu.make_async_copy(
        rch[0], out_hbm.at[last, pl.ds((nq - 2) * RCH, RCH)], sem_o[0]
    ).wait()
    pltpu.make_async_copy(
        rch[1], out_hbm.at[last, pl.ds((nq - 1) * RCH, RCH)], sem_o[1]
    ).wait()

    def xn_src(n, c):
        return xn_hbm.at[pl.ds(n * B + c * RCH, RCH)]

    pltpu.async_copy(xn_src(0, 0), rch[0], sem_x[0])

    def num_task(n, _):
        wb = plsc.load_gather(w_v, [n * D + d16])
        bb = plsc.load_gather(bias_v, [n * D + d16])
        r = n * D + d

        for c in range(B // RCH):
            p = c % 2
            b0 = c * RCH
            dst = out_hbm.at[r, pl.ds(b0, RCH)]
            pltpu.make_async_copy(xn_src(n, c), rch[p], sem_x[p]).wait()
            if c + 1 < B // RCH:
                nn_, ncc = n, c + 1
            else:
                nn_, ncc = jnp.minimum(n + 1, NN - 1), 0

            @plsc.parallel_loop(0, RCH, 16, unroll=8)
            def _(g):
                sl = pl.ds(g, 16)
                rch[p][sl] = rch[p][sl] * wb + bb
            pltpu.async_copy(rch[p], dst, sem_o[p])
            if c >= 1:
                pltpu.make_async_copy(rch[1 - p], dst, sem_o[1 - p]).wait()
            else:
                @pl.when(n > 0)
                def _():
                    pltpu.make_async_copy(rch[1 - p], dst, sem_o[1 - p]).wait()
            pltpu.async_copy(xn_src(nn_, ncc), rch[1 - p], sem_x[1 - p])
        return 0

    lax.fori_loop(0, NN, num_task, 0)

    lastn = (NN - 1) * D + d
    pltpu.make_async_copy(xn_src(NN - 1, 0), rch[0], sem_x[0]).wait()
    pltpu.make_async_copy(
        rch[1], out_hbm.at[lastn, pl.ds((nq - 1) * RCH, RCH)], sem_o[1]
    ).wait()


@jax.jit
def _run(emb_t, idx_t, xn_t, w1, bias1):
    mesh = plsc.VectorSubcoreMesh(
        core_axis_name="c", subcore_axis_name="s",
        num_cores=NUM_CORES, num_subcores=NUM_SUBCORES)
    f = pl.kernel(
        _body,
        out_type=jax.ShapeDtypeStruct(((NN + NC) * D, B), jnp.float32),
        mesh=mesh,
        scratch_types=[
            pltpu.VMEM((V,), jnp.float32),
            pltpu.VMEM((ICH,), jnp.int32),
            pltpu.VMEM((ICH,), jnp.int32),
            pltpu.VMEM((RCH,), jnp.float32),
            pltpu.VMEM((RCH,), jnp.float32),
            pltpu.VMEM((NN * D,), jnp.float32),
            pltpu.VMEM((NN * D,), jnp.float32),
            pltpu.SemaphoreType.DMA,
            pltpu.SemaphoreType.DMA,
            pltpu.SemaphoreType.DMA,
            pltpu.SemaphoreType.DMA,
            pltpu.SemaphoreType.DMA,
            pltpu.SemaphoreType.DMA,
        ],
        compiler_params=pltpu.CompilerParams(needs_layout_passes=False),
    )
    return f(emb_t, idx_t, xn_t, w1, bias1)


def kernel(x_numeric, x_categorical, W_num, b_num, emb):
    emb_t = jnp.transpose(emb, (0, 2, 1))
    idx_t = jnp.transpose(x_categorical[:, :, 0], (1, 0)).reshape(NC * B)
    xn_t = jnp.transpose(x_numeric[:, :, 0], (1, 0)).reshape(NN * B)
    w1 = W_num.reshape(NN * D)
    bias1 = b_num.reshape(NN * D)
    out = _run(emb_t, idx_t, xn_t, w1, bias1)
    return jnp.transpose(out, (1, 0)).reshape(B, 1, (NN + NC) * D)


# speedup vs baseline: 6.8238x; 1.1698x over previous
import jax
import jax.numpy as jnp
from jax import lax
from jax.experimental import pallas as pl
from jax.experimental.pallas import tpu as pltpu
from jax.experimental.pallas import tpu_sc as plsc

B = 16384
NN = 13
NC = 26
D = 32
V = 100000

NUM_CORES = 2
NUM_SUBCORES = 16
ICH = 8192
RCH = 4096
NCH = 2048
CPR = B // NCH
NKC = NN * CPR


def _body(emb_hbm, idx_hbm, xn_hbm, w_hbm, bias_hbm, out_hbm,
          plane_v, ich0_v, ich1_v, rch0_v, rch1_v, nch0_v, nch1_v,
          w_v, bias_v,
          sem_p, sem_i0, sem_i1, sem_o0, sem_o1,
          sem_x0, sem_x1, sem_n0, sem_n1):
    d = lax.axis_index("s") * NUM_CORES + lax.axis_index("c")
    ich = (ich0_v, ich1_v)
    rch = (rch0_v, rch1_v)
    nch = (nch0_v, nch1_v)
    sem_i = (sem_i0, sem_i1)
    sem_o = (sem_o0, sem_o1)
    sem_x = (sem_x0, sem_x1)
    sem_n = (sem_n0, sem_n1)

    pltpu.sync_copy(w_hbm, w_v)
    pltpu.sync_copy(bias_hbm, bias_v)
    d16 = jnp.full((16,), d, dtype=jnp.int32)

    def ich_src(f, h):
        return idx_hbm.at[pl.ds(f * B + h * ICH, ICH)]

    def xn_src(k):
        n = k // CPR
        c = k % CPR
        return xn_hbm.at[pl.ds(n * B + c * NCH, NCH)], n, c

    pltpu.async_copy(emb_hbm.at[0, d], plane_v, sem_p)
    pltpu.async_copy(ich_src(0, 0), ich[0], sem_i[0])
    src0, _, _ = xn_src(0)
    pltpu.async_copy(src0, nch[0], sem_x[0])

    def num_chunk(k, p):
        src, n, c = xn_src(k)
        wb = plsc.load_gather(w_v, [n * D + d16])
        bb = plsc.load_gather(bias_v, [n * D + d16])
        dst = out_hbm.at[n * D + d, pl.ds(c * NCH, NCH)]
        pltpu.make_async_copy(src, nch[p], sem_x[p]).wait()

        @plsc.parallel_loop(0, NCH, 16, unroll=8)
        def _(g):
            sl = pl.ds(g, 16)
            nch[p][sl] = nch[p][sl] * wb + bb

        pltpu.async_copy(nch[p], dst, sem_n[p])
        @pl.when(k >= 1)
        def _():
            pltpu.make_async_copy(nch[1 - p], dst, sem_n[1 - p]).wait()
        nsrc, _, _ = xn_src(jnp.minimum(k + 1, NKC - 1))
        pltpu.async_copy(nsrc, nch[1 - p], sem_x[1 - p])

    def cat_task(f, _):
        pltpu.make_async_copy(emb_hbm.at[f, d], plane_v, sem_p).wait()
        r = NN * D + f * D + d

        for h in range(B // ICH):
            pltpu.make_async_copy(ich_src(f, h), ich[h % 2], sem_i[h % 2]).wait()
            if h + 1 < B // ICH:
                nf, nh = f, h + 1
            else:
                nf, nh = jnp.minimum(f + 1, NC - 1), 0
            pltpu.async_copy(ich_src(nf, nh), ich[1 - h % 2], sem_i[1 - h % 2])

            for q in range(ICH // RCH):
                c = h * (ICH // RCH) + q
                p = c % 2
                b0 = c * RCH
                dst = out_hbm.at[r, pl.ds(b0, RCH)]
                if c >= 2:
                    pltpu.make_async_copy(rch[p], dst, sem_o[p]).wait()
                else:
                    @pl.when(f > 0)
                    def _():
                        pltpu.make_async_copy(rch[p], dst, sem_o[p]).wait()

                @plsc.parallel_loop(0, RCH, 16, unroll=8)
                def _(g):
                    rch[p][pl.ds(g, 16)] = plsc.load_gather(
                        plane_v, [ich[h % 2][pl.ds(q * RCH + g, 16)]])
                pltpu.async_copy(rch[p], dst, sem_o[p])

        pltpu.async_copy(emb_hbm.at[jnp.minimum(f + 1, NC - 1), d],
                         plane_v, sem_p)
        for j in range(NKC // NC):
            num_chunk(f * (NKC // NC) + j, j % 2)
        return 0

    lax.fori_loop(0, NC, cat_task, 0)

    last = NN * D + (NC - 1) * D + d
    nq = B // RCH
    pltpu.make_async_copy(emb_hbm.at[NC - 1, d], plane_v, sem_p).wait()
    pltpu.make_async_copy(ich_src(NC - 1, 0), ich[0], sem_i[0]).wait()
    pltpu.make_async_copy(
        rch[0], out_hbm.at[last, pl.ds((nq - 2) * RCH, RCH)], sem_o[0]
    ).wait()
    pltpu.make_async_copy(
        rch[1], out_hbm.at[last, pl.ds((nq - 1) * RCH, RCH)], sem_o[1]
    ).wait()
    lastn = (NN - 1) * D + d
    lsrc, _, _ = xn_src(NKC - 1)
    pltpu.make_async_copy(lsrc, nch[0], sem_x[0]).wait()
    pltpu.make_async_copy(
        nch[1], out_hbm.at[lastn, pl.ds((CPR - 1) * NCH, NCH)], sem_n[1]
    ).wait()


@jax.jit
def _run(emb_t, idx_t, xn_t, w1, bias1):
    mesh = plsc.VectorSubcoreMesh(
        core_axis_name="c", subcore_axis_name="s",
        num_cores=NUM_CORES, num_subcores=NUM_SUBCORES)
    f = pl.kernel(
        _body,
        out_type=jax.ShapeDtypeStruct(((NN + NC) * D, B), jnp.float32),
        mesh=mesh,
        scratch_types=[
            pltpu.VMEM((V,), jnp.float32),
            pltpu.VMEM((ICH,), jnp.int32),
            pltpu.VMEM((ICH,), jnp.int32),
            pltpu.VMEM((RCH,), jnp.float32),
            pltpu.VMEM((RCH,), jnp.float32),
            pltpu.VMEM((NCH,), jnp.float32),
            pltpu.VMEM((NCH,), jnp.float32),
            pltpu.VMEM((NN * D,), jnp.float32),
            pltpu.VMEM((NN * D,), jnp.float32),
        ] + [pltpu.SemaphoreType.DMA] * 9,
        compiler_params=pltpu.CompilerParams(needs_layout_passes=False),
    )
    return f(emb_t, idx_t, xn_t, w1, bias1)


def kernel(x_numeric, x_categorical, W_num, b_num, emb):
    emb_t = jnp.transpose(emb, (0, 2, 1))
    idx_t = jnp.transpose(x_categorical[:, :, 0], (1, 0)).reshape(NC * B)
    xn_t = jnp.transpose(x_numeric[:, :, 0], (1, 0)).reshape(NN * B)
    w1 = W_num.reshape(NN * D)
    bias1 = b_num.reshape(NN * D)
    out = _run(emb_t, idx_t, xn_t, w1, bias1)
    return jnp.transpose(out, (1, 0)).reshape(B, 1, (NN + NC) * D)
